# baseline (speedup 1.0000x reference)
import jax
import jax.numpy as jnp
from jax import lax
from jax.experimental import pallas as pl
from jax.experimental.pallas import tpu as pltpu
from jax.experimental.pallas import tpu_sc as plsc

VOCAB = 128
N_WORDS = 8192
WORD_LEN = 32

NUM_CORES = 2
NUM_SUBCORES = 16
LANES = 16
NUM_WORKERS = NUM_CORES * NUM_SUBCORES

ROWS_PER_WORKER = N_WORDS // NUM_WORKERS
GROUPS_PER_WORKER = ROWS_PER_WORKER // LANES


def _fofe_body(xt_hbm, ff_hbm, out_hbm, x_v, out_v, ff_v, in_sem, out_sem):
    wid = lax.axis_index("s") * NUM_CORES + lax.axis_index("c")
    row0 = wid * ROWS_PER_WORKER

    in_cp = pltpu.async_copy(
        xt_hbm.at[:, pl.ds(row0, ROWS_PER_WORKER)], x_v, in_sem)
    pltpu.sync_copy(ff_hbm, ff_v)
    ffv = ff_v[...]
    lane = lax.iota(jnp.int32, LANES)
    zeros = jnp.zeros((LANES,), jnp.float32)
    ones = jnp.ones((LANES,), jnp.float32)

    @plsc.parallel_loop(0, ROWS_PER_WORKER, 1, unroll=1)
    def zero_body(r):
        for t in range(VOCAB // LANES):
            out_v[r, pl.ds(t * LANES, LANES)] = zeros

    in_cp.wait()

    @plsc.parallel_loop(0, GROUPS_PER_WORKER, 1, unroll=1)
    def group_body(g):
        r0 = g * LANES
        rows = r0 + lane
        acc = ones
        for j in range(WORD_LEN):
            k = WORD_LEN - 1 - j
            c = x_v[k, pl.ds(r0, LANES)]
            m = c != 0
            plsc.addupdate_scatter(out_v, [rows, c], acc, mask=m)
            acc = jnp.where(m, acc * ffv, acc)

        pltpu.async_copy(
            out_v.at[pl.ds(r0, LANES), :],
            out_hbm.at[pl.ds(row0 + r0, LANES), :],
            out_sem)

    def drain_body(g, carry):
        r0 = g * LANES
        pltpu.make_async_copy(
            out_v.at[pl.ds(r0, LANES), :],
            out_hbm.at[pl.ds(row0 + r0, LANES), :],
            out_sem).wait()
        return carry

    lax.fori_loop(0, GROUPS_PER_WORKER, drain_body, 0)


@jax.jit
def kernel(x, forgetting_factor):
    xt = x.T
    ff_vec = jnp.broadcast_to(forgetting_factor.astype(jnp.float32), (LANES,))

    mesh = plsc.VectorSubcoreMesh(
        core_axis_name="c", subcore_axis_name="s",
        num_cores=NUM_CORES, num_subcores=NUM_SUBCORES,
    )
    return pl.kernel(
        _fofe_body,
        out_type=jax.ShapeDtypeStruct((N_WORDS, VOCAB), jnp.float32),
        mesh=mesh,
        compiler_params=pltpu.CompilerParams(needs_layout_passes=False),
        scratch_types=[
            pltpu.VMEM((WORD_LEN, ROWS_PER_WORKER), jnp.int32),
            pltpu.VMEM((ROWS_PER_WORKER, VOCAB), jnp.float32),
            pltpu.VMEM((LANES,), jnp.float32),
            pltpu.SemaphoreType.DMA,
            pltpu.SemaphoreType.DMA,
        ],
    )(xt, ff_vec)


# speedup vs baseline: 1.0319x; 1.0319x over previous
import jax
import jax.numpy as jnp
from jax import lax
from jax.experimental import pallas as pl
from jax.experimental.pallas import tpu as pltpu
from jax.experimental.pallas import tpu_sc as plsc

VOCAB = 128
N_WORDS = 8192
WORD_LEN = 32

NUM_CORES = 2
NUM_SUBCORES = 16
LANES = 16
NUM_WORKERS = NUM_CORES * NUM_SUBCORES

ROWS_PER_WORKER = N_WORDS // NUM_WORKERS
GROUPS_PER_WORKER = ROWS_PER_WORKER // LANES


def _fofe_body(xt_hbm, ff_hbm, out_hbm, x_v, out_v, ff_v, in_sem, out_sem):
    wid = lax.axis_index("s") * NUM_CORES + lax.axis_index("c")
    row0 = wid * ROWS_PER_WORKER

    in_cp = pltpu.async_copy(
        xt_hbm.at[:, pl.ds(row0, ROWS_PER_WORKER)], x_v, in_sem)
    pltpu.sync_copy(ff_hbm, ff_v.at[pl.ds(0, 1)])
    ffv = jnp.full((LANES,), ff_v[...][0])
    lane = lax.iota(jnp.int32, LANES)
    zeros = jnp.zeros((LANES,), jnp.float32)
    ones = jnp.ones((LANES,), jnp.float32)

    @plsc.parallel_loop(0, ROWS_PER_WORKER, 1, unroll=1)
    def zero_body(r):
        for t in range(VOCAB // LANES):
            out_v[r, pl.ds(t * LANES, LANES)] = zeros

    in_cp.wait()

    @plsc.parallel_loop(0, GROUPS_PER_WORKER, 1, unroll=1)
    def group_body(g):
        r0 = g * LANES
        rows = r0 + lane
        acc = ones
        for j in range(WORD_LEN):
            k = WORD_LEN - 1 - j
            c = x_v[k, pl.ds(r0, LANES)]
            m = c != 0
            plsc.addupdate_scatter(out_v, [rows, c], acc, mask=m)
            acc = jnp.where(m, acc * ffv, acc)

        pltpu.async_copy(
            out_v.at[pl.ds(r0, LANES), :],
            out_hbm.at[pl.ds(row0 + r0, LANES), :],
            out_sem)

    def drain_body(g, carry):
        r0 = g * LANES
        pltpu.make_async_copy(
            out_v.at[pl.ds(r0, LANES), :],
            out_hbm.at[pl.ds(row0 + r0, LANES), :],
            out_sem).wait()
        return carry

    lax.fori_loop(0, GROUPS_PER_WORKER, drain_body, 0)


@jax.jit
def kernel(x, forgetting_factor):
    xt = x.T
    ff_vec = forgetting_factor.astype(jnp.float32).reshape(1)

    mesh = plsc.VectorSubcoreMesh(
        core_axis_name="c", subcore_axis_name="s",
        num_cores=NUM_CORES, num_subcores=NUM_SUBCORES,
    )
    return pl.kernel(
        _fofe_body,
        out_type=jax.ShapeDtypeStruct((N_WORDS, VOCAB), jnp.float32),
        mesh=mesh,
        compiler_params=pltpu.CompilerParams(needs_layout_passes=False),
        scratch_types=[
            pltpu.VMEM((WORD_LEN, ROWS_PER_WORKER), jnp.int32),
            pltpu.VMEM((ROWS_PER_WORKER, VOCAB), jnp.float32),
            pltpu.VMEM((LANES,), jnp.float32),
            pltpu.SemaphoreType.DMA,
            pltpu.SemaphoreType.DMA,
        ],
    )(xt, ff_vec)
